# Initial kernel scaffold; baseline (speedup 1.0000x reference)
import jax
import jax.numpy as jnp
from jax.experimental import pallas as pl


def kernel(sounds, masks, start_token_ids, end_token_ids, embed_table, W_enc):
    raise NotImplementedError("write your pallas kernel here")


# speedup vs baseline: 2.1800x; 2.1800x over previous
import jax
import jax.numpy as jnp
from jax.experimental import pallas as pl
from jax.experimental.pallas import tpu as pltpu

_B, _T, _D_AUDIO = 16, 1500, 128
_D_MODEL = 1024
_N_START, _N_END = 4, 1
_T_OUT = _N_START + _T + _N_END


def _fused_body(start_ids_ref, end_ids_ref, sounds_ref, masks_ref, w_ref,
                embed_ref, out_ref, sems):
    copies = []
    for j in range(_N_START):
        idx = start_ids_ref[j]
        c = pltpu.make_async_copy(
            embed_ref.at[pl.ds(idx, 1), :],
            out_ref.at[0, pl.ds(j, 1), :],
            sems.at[j],
        )
        c.start()
        copies.append(c)
    for j in range(_N_END):
        idx = end_ids_ref[j]
        c = pltpu.make_async_copy(
            embed_ref.at[pl.ds(idx, 1), :],
            out_ref.at[0, pl.ds(_N_START + _T + j, 1), :],
            sems.at[_N_START + j],
        )
        c.start()
        copies.append(c)

    x = sounds_ref[0] * masks_ref[0, 0][:, None]
    out_ref[0, _N_START:_N_START + _T, :] = jnp.dot(
        x, w_ref[:, :], preferred_element_type=jnp.float32)

    for c in copies:
        c.wait()


def kernel(sounds, masks, start_token_ids, end_token_ids, embed_table, W_enc):
    masks3 = masks.reshape(_B, 1, _T)
    grid_spec = pltpu.PrefetchScalarGridSpec(
        num_scalar_prefetch=2,
        grid=(_B,),
        in_specs=[
            pl.BlockSpec((1, _T, _D_AUDIO), lambda b, *_: (b, 0, 0)),
            pl.BlockSpec((1, 1, _T), lambda b, *_: (b, 0, 0)),
            pl.BlockSpec((_D_AUDIO, _D_MODEL), lambda b, *_: (0, 0)),
            pl.BlockSpec(memory_space=pltpu.MemorySpace.HBM),
        ],
        out_specs=pl.BlockSpec((1, _T_OUT, _D_MODEL), lambda b, *_: (b, 0, 0)),
        scratch_shapes=[pltpu.SemaphoreType.DMA((_N_START + _N_END,))],
    )
    return pl.pallas_call(
        _fused_body,
        grid_spec=grid_spec,
        out_shape=jax.ShapeDtypeStruct((_B, _T_OUT, _D_MODEL), jnp.float32),
    )(start_token_ids.astype(jnp.int32), end_token_ids.astype(jnp.int32),
      sounds, masks3, W_enc, embed_table)


# speedup vs baseline: 2.1836x; 1.0016x over previous
import jax
import jax.numpy as jnp
from jax.experimental import pallas as pl
from jax.experimental.pallas import tpu as pltpu

_B, _T, _D_AUDIO = 16, 1500, 128
_D_MODEL = 1024
_N_START, _N_END = 4, 1
_T_OUT = _N_START + _T + _N_END


def _fused_body(start_ids_ref, end_ids_ref, sounds_ref, masks_ref, w_ref,
                embed_ref, out_ref, sems):
    copies = []
    for j in range(_N_START):
        idx = start_ids_ref[j]
        c = pltpu.make_async_copy(
            embed_ref.at[pl.ds(idx, 1), :],
            out_ref.at[0, pl.ds(j, 1), :],
            sems.at[j],
        )
        c.start()
        copies.append(c)
    for j in range(_N_END):
        idx = end_ids_ref[j]
        c = pltpu.make_async_copy(
            embed_ref.at[pl.ds(idx, 1), :],
            out_ref.at[0, pl.ds(_N_START + _T + j, 1), :],
            sems.at[_N_START + j],
        )
        c.start()
        copies.append(c)

    x = sounds_ref[0] * masks_ref[0, 0][:, None]
    out_ref[0, _N_START:_N_START + _T, :] = jnp.dot(
        x, w_ref[:, :], preferred_element_type=jnp.float32)

    for c in copies:
        c.wait()


def kernel(sounds, masks, start_token_ids, end_token_ids, embed_table, W_enc):
    masks3 = masks.reshape(_B, 1, _T)
    grid_spec = pltpu.PrefetchScalarGridSpec(
        num_scalar_prefetch=2,
        grid=(_B,),
        in_specs=[
            pl.BlockSpec((1, _T, _D_AUDIO), lambda b, *_: (b, 0, 0)),
            pl.BlockSpec((1, 1, _T), lambda b, *_: (b, 0, 0)),
            pl.BlockSpec((_D_AUDIO, _D_MODEL), lambda b, *_: (0, 0)),
            pl.BlockSpec(memory_space=pltpu.MemorySpace.HBM),
        ],
        out_specs=pl.BlockSpec((1, _T_OUT, _D_MODEL), lambda b, *_: (b, 0, 0)),
        scratch_shapes=[pltpu.SemaphoreType.DMA((_N_START + _N_END,))],
    )
    return pl.pallas_call(
        _fused_body,
        grid_spec=grid_spec,
        out_shape=jax.ShapeDtypeStruct((_B, _T_OUT, _D_MODEL), jnp.float32),
        compiler_params=pltpu.CompilerParams(
            dimension_semantics=("parallel",)),
    )(start_token_ids.astype(jnp.int32), end_token_ids.astype(jnp.int32),
      sounds, masks3, W_enc, embed_table)


# speedup vs baseline: 2.2229x; 1.0180x over previous
import jax
import jax.numpy as jnp
from jax.experimental import pallas as pl
from jax.experimental.pallas import tpu as pltpu

_B, _T, _D_AUDIO = 16, 1500, 128
_D_MODEL = 1024
_N_START, _N_END = 4, 1
_T_OUT = _N_START + _T + _N_END
_T_PAD = _N_START + _T


def _fused_body(start_ids_ref, end_ids_ref, sounds_ref, masks_ref, w_ref,
                embed_ref, out_ref, x_ref, emb_ref, sems):
    b = pl.program_id(0)

    @pl.when(b == 0)
    def _gather():
        copies = []
        for j in range(_N_START):
            c = pltpu.make_async_copy(
                embed_ref.at[pl.ds(start_ids_ref[j], 1), :],
                emb_ref.at[pl.ds(j, 1), :],
                sems.at[j],
            )
            c.start()
            copies.append(c)
        for j in range(_N_END):
            c = pltpu.make_async_copy(
                embed_ref.at[pl.ds(end_ids_ref[j], 1), :],
                emb_ref.at[pl.ds(_N_START + j, 1), :],
                sems.at[_N_START + j],
            )
            c.start()
            copies.append(c)
        x_ref[0:_N_START, :] = jnp.zeros((_N_START, _D_AUDIO), jnp.float32)
        for c in copies:
            c.wait()

    x_ref[_N_START:_T_PAD, :] = sounds_ref[0] * masks_ref[0, 0][:, None]
    out_ref[0, 0:_T_PAD, :] = jnp.dot(
        x_ref[:, :], w_ref[:, :], preferred_element_type=jnp.float32)
    out_ref[0, 0:_N_START, :] = emb_ref[0:_N_START, :]
    out_ref[0, _T_PAD:_T_OUT, :] = emb_ref[_N_START:_N_START + _N_END, :]


def kernel(sounds, masks, start_token_ids, end_token_ids, embed_table, W_enc):
    masks3 = masks.reshape(_B, 1, _T)
    grid_spec = pltpu.PrefetchScalarGridSpec(
        num_scalar_prefetch=2,
        grid=(_B,),
        in_specs=[
            pl.BlockSpec((1, _T, _D_AUDIO), lambda b, *_: (b, 0, 0)),
            pl.BlockSpec((1, 1, _T), lambda b, *_: (b, 0, 0)),
            pl.BlockSpec((_D_AUDIO, _D_MODEL), lambda b, *_: (0, 0)),
            pl.BlockSpec(memory_space=pltpu.MemorySpace.HBM),
        ],
        out_specs=pl.BlockSpec((1, _T_OUT, _D_MODEL), lambda b, *_: (b, 0, 0)),
        scratch_shapes=[
            pltpu.VMEM((_T_PAD, _D_AUDIO), jnp.float32),
            pltpu.VMEM((8, _D_MODEL), jnp.float32),
            pltpu.SemaphoreType.DMA((_N_START + _N_END,)),
        ],
    )
    return pl.pallas_call(
        _fused_body,
        grid_spec=grid_spec,
        out_shape=jax.ShapeDtypeStruct((_B, _T_OUT, _D_MODEL), jnp.float32),
        compiler_params=pltpu.CompilerParams(
            dimension_semantics=("arbitrary",)),
    )(start_token_ids.astype(jnp.int32), end_token_ids.astype(jnp.int32),
      sounds, masks3, W_enc, embed_table)
